# Initial kernel scaffold; baseline (speedup 1.0000x reference)
import jax
import jax.numpy as jnp
from jax.experimental import pallas as pl


def kernel(x, emb_weight):
    raise NotImplementedError("write your pallas kernel here")


# speedup vs baseline: 3.4008x; 3.4008x over previous
import jax
import jax.numpy as jnp
from jax.experimental import pallas as pl


def _copy_block(src_ref, out_ref):
    out_ref[...] = src_ref[...]


def kernel(x, emb_weight):
    seq_len = x.shape[1]
    dim = emb_weight.shape[1]
    block_rows = 512
    grid = (seq_len // block_rows,)
    out = pl.pallas_call(
        _copy_block,
        grid=grid,
        in_specs=[pl.BlockSpec((block_rows, dim), lambda i: (i, 0))],
        out_specs=pl.BlockSpec((block_rows, dim), lambda i: (i, 0)),
        out_shape=jax.ShapeDtypeStruct((seq_len, dim), emb_weight.dtype),
    )(emb_weight)
    return out[None, :, :]
